# Initial kernel scaffold; baseline (speedup 1.0000x reference)
import jax
import jax.numpy as jnp
from jax.experimental import pallas as pl


def kernel(x, edge_index, W, b):
    raise NotImplementedError("write your pallas kernel here")


# speedup vs baseline: 30.4862x; 30.4862x over previous
import functools

import jax
import jax.numpy as jnp
from jax import lax
from jax.experimental import pallas as pl
from jax.experimental.pallas import tpu as pltpu
from jax.experimental.pallas import tpu_sc as plsc

N = 10000
D = 128
E = 320000

NC = 2
NS = 16
NW = NC * NS
C = 128
K = 80
T = K * C
E_PAD = NW * T
N_PAD = 10240
R = N_PAD // NS
BLK = 1024

_MESH = plsc.VectorSubcoreMesh(core_axis_name="c", subcore_axis_name="s")


def _deg_body(dst_hbm, deg_out, dstv, ones_v, zbuf, acc):
    cid = lax.axis_index("c")
    sid = lax.axis_index("s")
    wid = sid * NC + cid

    for i in range(R // 16):
        zbuf[pl.ds(i * 16, 16)] = jnp.zeros((16,), jnp.float32)
    for i in range(C // 16):
        ones_v[pl.ds(i * 16, 16)] = jnp.ones((16,), jnp.float32)

    pltpu.sync_copy(dst_hbm.at[wid], dstv)
    pltpu.sync_copy(zbuf, acc.at[pl.ds(sid * R, R)])
    plsc.subcore_barrier()

    def step(j, _):
        pltpu.sync_copy(ones_v, acc.at[dstv.at[j]], add=True)
        return ()

    lax.fori_loop(0, K, step, ())

    plsc.subcore_barrier()
    pltpu.sync_copy(acc.at[pl.ds(sid * R, R)],
                    deg_out.at[cid, pl.ds(sid * R, R)])


_deg_kernel = functools.partial(
    pl.kernel,
    out_type=jax.ShapeDtypeStruct((NC, N_PAD), jnp.float32),
    mesh=_MESH,
    scratch_types=[
        pltpu.VMEM((K, C), jnp.int32),
        pltpu.VMEM((C,), jnp.float32),
        pltpu.VMEM((R,), jnp.float32),
        pltpu.VMEM_SHARED((N_PAD,), jnp.float32),
    ],
)(_deg_body)


def _gs_body(g_hbm, src_hbm, dst_hbm, parts, srcv, dstv, rows0,
             sem0, acc):
    cid = lax.axis_index("c")
    sid = lax.axis_index("s")
    wid = sid * NC + cid

    pltpu.sync_copy(src_hbm.at[wid], srcv)
    pltpu.sync_copy(dst_hbm.at[wid], dstv)

    def zb(i, _):
        for l in range(D // 16):
            rows0[i, pl.ds(l * 16, 16)] = jnp.zeros((16,), jnp.float32)
        return ()

    lax.fori_loop(0, C, zb, ())
    for kblk in range(R // C):
        pltpu.sync_copy(rows0, acc.at[pl.ds(sid * R + kblk * C, C)])
    plsc.subcore_barrier()

    def step(j, _):
        pltpu.async_copy(g_hbm.at[srcv.at[j]], rows0, sem0).wait()
        pltpu.sync_copy(rows0, acc.at[dstv.at[j]], add=True)
        return ()

    lax.fori_loop(0, K, step, ())

    plsc.subcore_barrier()
    for kblk in range(R // C):
        pltpu.sync_copy(acc.at[pl.ds(sid * R + kblk * C, C)],
                        parts.at[cid, pl.ds(sid * R + kblk * C, C)])


_gs_kernel = functools.partial(
    pl.kernel,
    out_type=jax.ShapeDtypeStruct((NC, N_PAD, D), jnp.float32),
    mesh=_MESH,
    scratch_types=[
        pltpu.VMEM((K, C), jnp.int32),
        pltpu.VMEM((K, C), jnp.int32),
        pltpu.VMEM((C, D), jnp.float32),
        pltpu.SemaphoreType.DMA,
        pltpu.VMEM_SHARED((N_PAD, D), jnp.float32),
    ],
)(_gs_body)


def _mm_body(d_ref, x_ref, w_ref, g_ref):
    deg = d_ref[0] + d_ref[1] + 1.0
    dinv = lax.rsqrt(deg)
    h = jnp.dot(x_ref[...], w_ref[...], preferred_element_type=jnp.float32)
    g_ref[...] = h * dinv


def _matmul_scaled(deg3, x_pad, W):
    return pl.pallas_call(
        _mm_body,
        grid=(N_PAD // BLK,),
        in_specs=[
            pl.BlockSpec((NC, BLK, 1), lambda i: (0, i, 0)),
            pl.BlockSpec((BLK, D), lambda i: (i, 0)),
            pl.BlockSpec((D, D), lambda i: (0, 0)),
        ],
        out_specs=pl.BlockSpec((BLK, D), lambda i: (i, 0)),
        out_shape=jax.ShapeDtypeStruct((N_PAD, D), jnp.float32),
    )(deg3, x_pad, W)


def _fin_body(d_ref, p_ref, g_ref, b_ref, o_ref):
    deg = d_ref[0] + d_ref[1] + 1.0
    dinv = lax.rsqrt(deg)
    s = p_ref[0] + p_ref[1] + g_ref[...]
    o_ref[...] = s * dinv + b_ref[...]


def _finalize(deg3, parts, g, b2):
    return pl.pallas_call(
        _fin_body,
        grid=(N_PAD // BLK,),
        in_specs=[
            pl.BlockSpec((NC, BLK, 1), lambda i: (0, i, 0)),
            pl.BlockSpec((NC, BLK, D), lambda i: (0, i, 0)),
            pl.BlockSpec((BLK, D), lambda i: (i, 0)),
            pl.BlockSpec((1, D), lambda i: (0, 0)),
        ],
        out_specs=pl.BlockSpec((BLK, D), lambda i: (i, 0)),
        out_shape=jax.ShapeDtypeStruct((N_PAD, D), jnp.float32),
    )(deg3, parts, g, b2)


def kernel(x, edge_index, W, b):
    pad_n = E_PAD - E
    fill = N + (jnp.arange(pad_n, dtype=jnp.int32) % (N_PAD - N))
    src_p = jnp.concatenate([edge_index[0], fill]).reshape(NW, K, C)
    dst_p = jnp.concatenate([edge_index[1], fill]).reshape(NW, K, C)
    x_pad = jnp.pad(x, ((0, N_PAD - N), (0, 0)))

    deg_parts = _deg_kernel(dst_p)
    deg3 = deg_parts.reshape(NC, N_PAD, 1)
    g = _matmul_scaled(deg3, x_pad, W)
    parts = _gs_kernel(g, src_p, dst_p)
    out = _finalize(deg3, parts, g, b.reshape(1, D))
    return out[:N]


# speedup vs baseline: 41.8052x; 1.3713x over previous
import functools

import jax
import jax.numpy as jnp
from jax import lax
from jax.experimental import pallas as pl
from jax.experimental.pallas import tpu as pltpu
from jax.experimental.pallas import tpu_sc as plsc

N = 10000
D = 128
E = 320000

NC = 2
NS = 16
NW = NC * NS
C = 128
K = 80
T = K * C
E_PAD = NW * T
N_PAD = 10240
R = N_PAD // NS
BLK = 1024

_MESH = plsc.VectorSubcoreMesh(core_axis_name="c", subcore_axis_name="s")


def _deg_body(dst_hbm, deg_out, dstv, ones_v, zbuf, acc):
    cid = lax.axis_index("c")
    sid = lax.axis_index("s")
    wid = sid * NC + cid

    for i in range(R // 16):
        zbuf[pl.ds(i * 16, 16)] = jnp.zeros((16,), jnp.float32)
    for i in range(C // 16):
        ones_v[pl.ds(i * 16, 16)] = jnp.ones((16,), jnp.float32)

    pltpu.sync_copy(dst_hbm.at[wid], dstv)
    pltpu.sync_copy(zbuf, acc.at[pl.ds(sid * R, R)])
    plsc.subcore_barrier()

    def step(j, _):
        pltpu.sync_copy(ones_v, acc.at[dstv.at[j]], add=True)
        return ()

    lax.fori_loop(0, K, step, ())

    plsc.subcore_barrier()
    pltpu.sync_copy(acc.at[pl.ds(sid * R, R)],
                    deg_out.at[cid, pl.ds(sid * R, R)])


_deg_kernel = functools.partial(
    pl.kernel,
    out_type=jax.ShapeDtypeStruct((NC, N_PAD), jnp.float32),
    mesh=_MESH,
    scratch_types=[
        pltpu.VMEM((K, C), jnp.int32),
        pltpu.VMEM((C,), jnp.float32),
        pltpu.VMEM((R,), jnp.float32),
        pltpu.VMEM_SHARED((N_PAD,), jnp.float32),
    ],
)(_deg_body)


IB = 16
NB = K // IB


def _gs_body(g_hbm, src_hbm, dst_hbm, parts, srcv, dstv, rows,
             semi, semg0, semg1, acc):
    cid = lax.axis_index("c")
    sid = lax.axis_index("s")
    wid = sid * NC + cid
    semg = (semg0, semg1)

    idx_pend = {}

    def issue_idx(b):
        slot = b % 3
        idx_pend[b] = (
            pltpu.async_copy(src_hbm.at[wid, pl.ds(b * IB, IB)],
                             srcv.at[slot], semi),
            pltpu.async_copy(dst_hbm.at[wid, pl.ds(b * IB, IB)],
                             dstv.at[slot], semi),
        )

    issue_idx(0)

    def zb(i, _):
        for l in range(D // 16):
            rows[0, i, pl.ds(l * 16, 16)] = jnp.zeros((16,), jnp.float32)
        return ()

    lax.fori_loop(0, C, zb, ())
    for kblk in range(R // C):
        pltpu.sync_copy(rows.at[0], acc.at[pl.ds(sid * R + kblk * C, C)])
    plsc.subcore_barrier()

    def enter_block(b):
        s, d2 = idx_pend.pop(b)
        s.wait()
        d2.wait()
        if b + 1 < NB:
            issue_idx(b + 1)

    def gather(q):
        b, jj = divmod(q, IB)
        return pltpu.async_copy(g_hbm.at[srcv.at[b % 3, jj]],
                                rows.at[q % 2], semg[q % 2])

    enter_block(0)
    pending = gather(0)
    for q in range(1, K + 1):
        if q < K:
            b, jj = divmod(q, IB)
            if jj == 0:
                enter_block(b)
            nxt = gather(q)
        pending.wait()
        bp, jp = divmod(q - 1, IB)
        pltpu.sync_copy(rows.at[(q - 1) % 2],
                        acc.at[dstv.at[bp % 3, jp]], add=True)
        if q < K:
            pending = nxt

    plsc.subcore_barrier()
    for kblk in range(R // C):
        pltpu.sync_copy(acc.at[pl.ds(sid * R + kblk * C, C)],
                        parts.at[cid, pl.ds(sid * R + kblk * C, C)])


_gs_kernel = functools.partial(
    pl.kernel,
    out_type=jax.ShapeDtypeStruct((NC, N_PAD, D), jnp.float32),
    mesh=_MESH,
    scratch_types=[
        pltpu.VMEM((3, IB, C), jnp.int32),
        pltpu.VMEM((3, IB, C), jnp.int32),
        pltpu.VMEM((2, C, D), jnp.float32),
        pltpu.SemaphoreType.DMA,
        pltpu.SemaphoreType.DMA,
        pltpu.SemaphoreType.DMA,
        pltpu.VMEM_SHARED((N_PAD, D), jnp.float32),
    ],
)(_gs_body)


def _mm_body(d_ref, x_ref, w_ref, g_ref):
    deg = d_ref[0] + d_ref[1] + 1.0
    dinv = lax.rsqrt(deg)
    h = jnp.dot(x_ref[...], w_ref[...], preferred_element_type=jnp.float32)
    g_ref[...] = h * dinv


def _matmul_scaled(deg3, x_pad, W):
    return pl.pallas_call(
        _mm_body,
        grid=(N_PAD // BLK,),
        in_specs=[
            pl.BlockSpec((NC, BLK, 1), lambda i: (0, i, 0)),
            pl.BlockSpec((BLK, D), lambda i: (i, 0)),
            pl.BlockSpec((D, D), lambda i: (0, 0)),
        ],
        out_specs=pl.BlockSpec((BLK, D), lambda i: (i, 0)),
        out_shape=jax.ShapeDtypeStruct((N_PAD, D), jnp.float32),
    )(deg3, x_pad, W)


def _fin_body(d_ref, p_ref, g_ref, b_ref, o_ref):
    deg = d_ref[0] + d_ref[1] + 1.0
    dinv = lax.rsqrt(deg)
    s = p_ref[0] + p_ref[1] + g_ref[...]
    o_ref[...] = s * dinv + b_ref[...]


def _finalize(deg3, parts, g, b2):
    return pl.pallas_call(
        _fin_body,
        grid=(N_PAD // BLK,),
        in_specs=[
            pl.BlockSpec((NC, BLK, 1), lambda i: (0, i, 0)),
            pl.BlockSpec((NC, BLK, D), lambda i: (0, i, 0)),
            pl.BlockSpec((BLK, D), lambda i: (i, 0)),
            pl.BlockSpec((1, D), lambda i: (0, 0)),
        ],
        out_specs=pl.BlockSpec((BLK, D), lambda i: (i, 0)),
        out_shape=jax.ShapeDtypeStruct((N_PAD, D), jnp.float32),
    )(deg3, parts, g, b2)


def kernel(x, edge_index, W, b):
    pad_n = E_PAD - E
    fill = N + (jnp.arange(pad_n, dtype=jnp.int32) % (N_PAD - N))
    src_p = jnp.concatenate([edge_index[0], fill]).reshape(NW, K, C)
    dst_p = jnp.concatenate([edge_index[1], fill]).reshape(NW, K, C)
    x_pad = jnp.pad(x, ((0, N_PAD - N), (0, 0)))

    deg_parts = _deg_kernel(dst_p)
    deg3 = deg_parts.reshape(NC, N_PAD, 1)
    g = _matmul_scaled(deg3, x_pad, W)
    parts = _gs_kernel(g, src_p, dst_p)
    out = _finalize(deg3, parts, g, b.reshape(1, D))
    return out[:N]


# speedup vs baseline: 43.4253x; 1.0388x over previous
import functools

import jax
import jax.numpy as jnp
from jax import lax
from jax.experimental import pallas as pl
from jax.experimental.pallas import tpu as pltpu
from jax.experimental.pallas import tpu_sc as plsc

N = 10000
D = 128
E = 320000

NC = 2
NS = 16
NW = NC * NS
C = 128
K = 80
T = K * C
E_PAD = NW * T
N_PAD = 10240
R = N_PAD // NS
BLK = 1000

_MESH = plsc.VectorSubcoreMesh(core_axis_name="c", subcore_axis_name="s")


def _deg_body(dst_hbm, deg_out, dstv, ones_v, zbuf, acc):
    cid = lax.axis_index("c")
    sid = lax.axis_index("s")
    wid = sid * NC + cid

    for i in range(R // 16):
        zbuf[pl.ds(i * 16, 16)] = jnp.zeros((16,), jnp.float32)
    for i in range(C // 16):
        ones_v[pl.ds(i * 16, 16)] = jnp.ones((16,), jnp.float32)

    pltpu.sync_copy(dst_hbm.at[wid], dstv)
    pltpu.sync_copy(zbuf, acc.at[pl.ds(sid * R, R)])
    plsc.subcore_barrier()

    def step(j, _):
        pltpu.sync_copy(ones_v, acc.at[dstv.at[j]], add=True)
        return ()

    lax.fori_loop(0, K, step, ())

    plsc.subcore_barrier()
    pltpu.sync_copy(acc.at[pl.ds(sid * R, R)],
                    deg_out.at[cid, pl.ds(sid * R, R)])


_deg_kernel = functools.partial(
    pl.kernel,
    out_type=jax.ShapeDtypeStruct((NC, N_PAD), jnp.float32),
    mesh=_MESH,
    scratch_types=[
        pltpu.VMEM((K, C), jnp.int32),
        pltpu.VMEM((C,), jnp.float32),
        pltpu.VMEM((R,), jnp.float32),
        pltpu.VMEM_SHARED((N_PAD,), jnp.float32),
    ],
)(_deg_body)


IB = 16
NB = K // IB


def _gs_body(g_hbm, src_hbm, dst_hbm, parts, srcv, dstv, rows,
             semi, semg0, semg1, acc):
    cid = lax.axis_index("c")
    sid = lax.axis_index("s")
    wid = sid * NC + cid
    semg = (semg0, semg1)

    idx_pend = {}

    def issue_idx(b):
        slot = b % 3
        idx_pend[b] = (
            pltpu.async_copy(src_hbm.at[wid, pl.ds(b * IB, IB)],
                             srcv.at[slot], semi),
            pltpu.async_copy(dst_hbm.at[wid, pl.ds(b * IB, IB)],
                             dstv.at[slot], semi),
        )

    issue_idx(0)

    def zb(i, _):
        for l in range(D // 16):
            rows[0, i, pl.ds(l * 16, 16)] = jnp.zeros((16,), jnp.float32)
        return ()

    lax.fori_loop(0, C, zb, ())
    for kblk in range(R // C):
        pltpu.sync_copy(rows.at[0], acc.at[pl.ds(sid * R + kblk * C, C)])
    plsc.subcore_barrier()

    def enter_block(b):
        s, d2 = idx_pend.pop(b)
        s.wait()
        d2.wait()
        if b + 1 < NB:
            issue_idx(b + 1)

    def gather(q):
        b, jj = divmod(q, IB)
        return pltpu.async_copy(g_hbm.at[srcv.at[b % 3, jj]],
                                rows.at[q % 2], semg[q % 2])

    enter_block(0)
    pending = gather(0)
    for q in range(1, K + 1):
        if q < K:
            b, jj = divmod(q, IB)
            if jj == 0:
                enter_block(b)
            nxt = gather(q)
        pending.wait()
        bp, jp = divmod(q - 1, IB)
        pltpu.sync_copy(rows.at[(q - 1) % 2],
                        acc.at[dstv.at[bp % 3, jp]], add=True)
        if q < K:
            pending = nxt

    plsc.subcore_barrier()
    for kblk in range(R // C):
        pltpu.sync_copy(acc.at[pl.ds(sid * R + kblk * C, C)],
                        parts.at[cid, pl.ds(sid * R + kblk * C, C)])


_gs_kernel = functools.partial(
    pl.kernel,
    out_type=jax.ShapeDtypeStruct((NC, N_PAD, D), jnp.float32),
    mesh=_MESH,
    scratch_types=[
        pltpu.VMEM((3, IB, C), jnp.int32),
        pltpu.VMEM((3, IB, C), jnp.int32),
        pltpu.VMEM((2, C, D), jnp.float32),
        pltpu.SemaphoreType.DMA,
        pltpu.SemaphoreType.DMA,
        pltpu.SemaphoreType.DMA,
        pltpu.VMEM_SHARED((N_PAD, D), jnp.float32),
    ],
)(_gs_body)


def _mm_body(d_ref, x_ref, w_ref, g_ref):
    deg = d_ref[0] + d_ref[1] + 1.0
    dinv = lax.rsqrt(deg)
    h = jnp.dot(x_ref[...], w_ref[...], preferred_element_type=jnp.float32)
    g_ref[...] = h * dinv


def _matmul_scaled(deg3, x, W):
    return pl.pallas_call(
        _mm_body,
        grid=(N // BLK,),
        in_specs=[
            pl.BlockSpec((NC, BLK, 1), lambda i: (0, i, 0)),
            pl.BlockSpec((BLK, D), lambda i: (i, 0)),
            pl.BlockSpec((D, D), lambda i: (0, 0)),
        ],
        out_specs=pl.BlockSpec((BLK, D), lambda i: (i, 0)),
        out_shape=jax.ShapeDtypeStruct((N, D), jnp.float32),
    )(deg3, x, W)


def _fin_body(d_ref, p_ref, g_ref, b_ref, o_ref):
    deg = d_ref[0] + d_ref[1] + 1.0
    dinv = lax.rsqrt(deg)
    s = p_ref[0] + p_ref[1] + g_ref[...]
    o_ref[...] = s * dinv + b_ref[...]


def _finalize(deg3, parts, g, b2):
    return pl.pallas_call(
        _fin_body,
        grid=(N // BLK,),
        in_specs=[
            pl.BlockSpec((NC, BLK, 1), lambda i: (0, i, 0)),
            pl.BlockSpec((NC, BLK, D), lambda i: (0, i, 0)),
            pl.BlockSpec((BLK, D), lambda i: (i, 0)),
            pl.BlockSpec((1, D), lambda i: (0, 0)),
        ],
        out_specs=pl.BlockSpec((BLK, D), lambda i: (i, 0)),
        out_shape=jax.ShapeDtypeStruct((N, D), jnp.float32),
    )(deg3, parts, g, b2)


def kernel(x, edge_index, W, b):
    pad_n = E_PAD - E
    ar = jnp.arange(pad_n, dtype=jnp.int32)
    fill_src = (ar * 73) % N
    fill_dst = N + ar % (N_PAD - N)
    src_p = jnp.concatenate([edge_index[0], fill_src]).reshape(NW, K, C)
    dst_p = jnp.concatenate([edge_index[1], fill_dst]).reshape(NW, K, C)

    deg_parts = _deg_kernel(dst_p)
    deg3 = deg_parts.reshape(NC, N_PAD, 1)
    g = _matmul_scaled(deg3, x, W)
    parts = _gs_kernel(g, src_p, dst_p)
    return _finalize(deg3, parts, g, b.reshape(1, D))


# speedup vs baseline: 46.0455x; 1.0603x over previous
import functools

import jax
import jax.numpy as jnp
from jax import lax
from jax.experimental import pallas as pl
from jax.experimental.pallas import tpu as pltpu
from jax.experimental.pallas import tpu_sc as plsc

N = 10000
D = 128
E = 320000

NC = 2
NS = 16
NW = NC * NS
C = 128
NCH = E // C
K = 79
N_PAD = 10240
R = N_PAD // NS
BLK = 1000

IB = 16
NB = (K + IB - 1) // IB

_MESH = plsc.VectorSubcoreMesh(core_axis_name="c", subcore_axis_name="s")


def _chunk_off(wid, i):
    cstart = 78 * wid + jnp.minimum(wid, 4)
    if i == K - 1:
        return jnp.where(wid < 4, (cstart + (K - 1)) * C, 0)
    return (cstart + i) * C


def _dummy_fill(ref, row, wid):
    iota = lax.iota(jnp.int32, 16)
    for kk in range(C // 16):
        src_vals = (wid * 997 + kk * 1009 + iota * 131) % N
        dst_vals = N + (wid * 16 + kk * 16 + iota) % (N_PAD - N)
        ref[row, 0, pl.ds(kk * 16, 16)] = src_vals
        ref[row, 1, pl.ds(kk * 16, 16)] = dst_vals


def _deg_body(edge_hbm, deg_out, edg, ones_v, zbuf, semi, acc):
    cid = lax.axis_index("c")
    sid = lax.axis_index("s")
    wid = sid * NC + cid

    descs = [
        pltpu.async_copy(edge_hbm.at[:, pl.ds(_chunk_off(wid, q), C)],
                         edg.at[q], semi)
        for q in range(K)
    ]

    for i in range(R // 16):
        zbuf[pl.ds(i * 16, 16)] = jnp.zeros((16,), jnp.float32)
    for i in range(C // 16):
        ones_v[pl.ds(i * 16, 16)] = jnp.ones((16,), jnp.float32)

    pltpu.sync_copy(zbuf, acc.at[pl.ds(sid * R, R)])
    plsc.subcore_barrier()

    for d2 in descs:
        d2.wait()

    @pl.when(wid >= 4)
    def _():
        _dummy_fill(edg, K - 1, wid)

    for q in range(K):
        pltpu.sync_copy(ones_v, acc.at[edg.at[q, 1]], add=True)

    plsc.subcore_barrier()
    pltpu.sync_copy(acc.at[pl.ds(sid * R, R)],
                    deg_out.at[cid, pl.ds(sid * R, R)])


_deg_kernel = functools.partial(
    pl.kernel,
    out_type=jax.ShapeDtypeStruct((NC, N_PAD), jnp.float32),
    mesh=_MESH,
    scratch_types=[
        pltpu.VMEM((K, 2, C), jnp.int32),
        pltpu.VMEM((C,), jnp.float32),
        pltpu.VMEM((R,), jnp.float32),
        pltpu.SemaphoreType.DMA,
        pltpu.VMEM_SHARED((N_PAD,), jnp.float32),
    ],
)(_deg_body)


def _gs_body(g_hbm, edge_hbm, parts, edg, rows,
             semi, semg0, semg1, semf, acc):
    cid = lax.axis_index("c")
    sid = lax.axis_index("s")
    wid = sid * NC + cid
    semg = (semg0, semg1)

    idx_pend = {}

    def issue_idx(b):
        slot = b % 3
        nrows = IB if b < NB - 1 else (K - b * IB)
        idx_pend[b] = [
            pltpu.async_copy(
                edge_hbm.at[:, pl.ds(_chunk_off(wid, b * IB + i), C)],
                edg.at[slot, i], semi)
            for i in range(nrows)
        ]

    def enter_block(b):
        for d2 in idx_pend.pop(b):
            d2.wait()
        if b == NB - 1:
            @pl.when(wid >= 4)
            def _():
                _dummy_fill(edg.at[b % 3], (K - 1) % IB, wid)
        if b + 1 < NB:
            issue_idx(b + 1)

    def gather(q):
        b, jj = divmod(q, IB)
        return pltpu.async_copy(g_hbm.at[edg.at[b % 3, jj, 0]],
                                rows.at[q % 2], semg[q % 2])

    issue_idx(0)

    def zb(i, _):
        for l in range(D // 16):
            rows[0, i, pl.ds(l * 16, 16)] = jnp.zeros((16,), jnp.float32)
        return ()

    lax.fori_loop(0, C, zb, ())
    for kblk in range(R // C):
        pltpu.sync_copy(rows.at[0], acc.at[pl.ds(sid * R + kblk * C, C)])

    enter_block(0)
    g_desc = {0: gather(0), 1: gather(1)}
    plsc.subcore_barrier()

    for q in range(K):
        t = q + 2
        if t < K and t % IB == 0:
            enter_block(t // IB)
        g_desc.pop(q).wait()
        b, jj = divmod(q, IB)
        pltpu.sync_copy(rows.at[q % 2], acc.at[edg.at[b % 3, jj, 1]],
                        add=True)
        if t < K:
            g_desc[t] = gather(t)

    plsc.subcore_barrier()
    flush = [
        pltpu.async_copy(acc.at[pl.ds(sid * R + kblk * C, C)],
                         parts.at[cid, pl.ds(sid * R + kblk * C, C)], semf)
        for kblk in range(R // C)
    ]
    for d2 in flush:
        d2.wait()


_gs_kernel = functools.partial(
    pl.kernel,
    out_type=jax.ShapeDtypeStruct((NC, N_PAD, D), jnp.float32),
    mesh=_MESH,
    scratch_types=[
        pltpu.VMEM((3, IB, 2, C), jnp.int32),
        pltpu.VMEM((2, C, D), jnp.float32),
        pltpu.SemaphoreType.DMA,
        pltpu.SemaphoreType.DMA,
        pltpu.SemaphoreType.DMA,
        pltpu.SemaphoreType.DMA,
        pltpu.VMEM_SHARED((N_PAD, D), jnp.float32),
    ],
)(_gs_body)


def _mm_body(d_ref, x_ref, w_ref, g_ref):
    deg = d_ref[0] + d_ref[1] + 1.0
    dinv = lax.rsqrt(deg)
    h = jnp.dot(x_ref[...], w_ref[...], preferred_element_type=jnp.float32)
    g_ref[...] = h * dinv


def _matmul_scaled(deg3, x, W):
    return pl.pallas_call(
        _mm_body,
        grid=(N // BLK,),
        in_specs=[
            pl.BlockSpec((NC, BLK, 1), lambda i: (0, i, 0)),
            pl.BlockSpec((BLK, D), lambda i: (i, 0)),
            pl.BlockSpec((D, D), lambda i: (0, 0)),
        ],
        out_specs=pl.BlockSpec((BLK, D), lambda i: (i, 0)),
        out_shape=jax.ShapeDtypeStruct((N, D), jnp.float32),
    )(deg3, x, W)


def _fin_body(d_ref, p_ref, g_ref, b_ref, o_ref):
    deg = d_ref[0] + d_ref[1] + 1.0
    dinv = lax.rsqrt(deg)
    s = p_ref[0] + p_ref[1] + g_ref[...]
    o_ref[...] = s * dinv + b_ref[...]


def _finalize(deg3, parts, g, b2):
    return pl.pallas_call(
        _fin_body,
        grid=(N // BLK,),
        in_specs=[
            pl.BlockSpec((NC, BLK, 1), lambda i: (0, i, 0)),
            pl.BlockSpec((NC, BLK, D), lambda i: (0, i, 0)),
            pl.BlockSpec((BLK, D), lambda i: (i, 0)),
            pl.BlockSpec((1, D), lambda i: (0, 0)),
        ],
        out_specs=pl.BlockSpec((BLK, D), lambda i: (i, 0)),
        out_shape=jax.ShapeDtypeStruct((N, D), jnp.float32),
    )(deg3, parts, g, b2)


def kernel(x, edge_index, W, b):
    deg_parts = _deg_kernel(edge_index)
    deg3 = deg_parts.reshape(NC, N_PAD, 1)
    g = _matmul_scaled(deg3, x, W)
    parts = _gs_kernel(g, edge_index)
    return _finalize(deg3, parts, g, b.reshape(1, D))


# speedup vs baseline: 49.5684x; 1.0765x over previous
import functools

import jax
import jax.numpy as jnp
from jax import lax
from jax.experimental import pallas as pl
from jax.experimental.pallas import tpu as pltpu
from jax.experimental.pallas import tpu_sc as plsc

N = 10000
D = 128
E = 320000

NC = 2
NS = 16
NW = NC * NS
C = 128
NCH = E // C
K = 79
N_PAD = 10240
R = N_PAD // NS
BLK = 1024

IB = 16
NB = (K + IB - 1) // IB

_MESH = plsc.VectorSubcoreMesh(core_axis_name="c", subcore_axis_name="s")


def _chunk_off(wid, i):
    cstart = 78 * wid + jnp.minimum(wid, 4)
    if i == K - 1:
        return jnp.where(wid < 4, (cstart + (K - 1)) * C, 0)
    return (cstart + i) * C


def _dummy_fill(ref, row, wid):
    iota = lax.iota(jnp.int32, 16)
    for kk in range(C // 16):
        src_vals = (wid * 997 + kk * 1009 + iota * 131) % N
        dst_vals = N + (wid * 16 + kk * 16 + iota) % (N_PAD - N)
        ref[row, 0, pl.ds(kk * 16, 16)] = src_vals
        ref[row, 1, pl.ds(kk * 16, 16)] = dst_vals


def _deg_body(edge_hbm, deg_out, edg, ones_v, zbuf, semi, sems, acc):
    cid = lax.axis_index("c")
    sid = lax.axis_index("s")
    wid = sid * NC + cid

    descs = [
        pltpu.async_copy(edge_hbm.at[:, pl.ds(_chunk_off(wid, q), C)],
                         edg.at[q], semi)
        for q in range(K)
    ]

    for i in range(R // 16):
        zbuf[pl.ds(i * 16, 16)] = jnp.zeros((16,), jnp.float32)
    for i in range(C // 16):
        ones_v[pl.ds(i * 16, 16)] = jnp.ones((16,), jnp.float32)

    pltpu.sync_copy(zbuf, acc.at[pl.ds(sid * R, R)])
    plsc.subcore_barrier()

    for d2 in descs:
        d2.wait()

    @pl.when(wid >= 4)
    def _():
        _dummy_fill(edg, K - 1, wid)

    sdescs = [
        pltpu.async_copy(ones_v, acc.at[edg.at[q, 1]], sems, add=True)
        for q in range(K)
    ]
    for d2 in sdescs:
        d2.wait()

    plsc.subcore_barrier()
    pltpu.sync_copy(acc.at[pl.ds(sid * R, R)],
                    deg_out.at[cid, pl.ds(sid * R, R)])


_deg_kernel = functools.partial(
    pl.kernel,
    out_type=jax.ShapeDtypeStruct((NC, N_PAD), jnp.float32),
    mesh=_MESH,
    scratch_types=[
        pltpu.VMEM((K, 2, C), jnp.int32),
        pltpu.VMEM((C,), jnp.float32),
        pltpu.VMEM((R,), jnp.float32),
        pltpu.SemaphoreType.DMA,
        pltpu.SemaphoreType.DMA,
        pltpu.VMEM_SHARED((N_PAD,), jnp.float32),
    ],
)(_deg_body)


def _gs_body(g_hbm, edge_hbm, parts, edg, rows,
             semi, semg0, semg1, semf, acc):
    cid = lax.axis_index("c")
    sid = lax.axis_index("s")
    wid = sid * NC + cid
    semg = (semg0, semg1)

    idx_pend = {}

    def issue_idx(b):
        slot = b % 3
        nrows = IB if b < NB - 1 else (K - b * IB)
        idx_pend[b] = [
            pltpu.async_copy(
                edge_hbm.at[:, pl.ds(_chunk_off(wid, b * IB + i), C)],
                edg.at[slot, i], semi)
            for i in range(nrows)
        ]

    def enter_block(b):
        for d2 in idx_pend.pop(b):
            d2.wait()
        if b == NB - 1:
            @pl.when(wid >= 4)
            def _():
                _dummy_fill(edg.at[b % 3], (K - 1) % IB, wid)
        if b + 1 < NB:
            issue_idx(b + 1)

    def gather(q):
        b, jj = divmod(q, IB)
        return pltpu.async_copy(g_hbm.at[edg.at[b % 3, jj, 0]],
                                rows.at[q % 2], semg[q % 2])

    issue_idx(0)

    def zb(i, _):
        for l in range(D // 16):
            rows[0, i, pl.ds(l * 16, 16)] = jnp.zeros((16,), jnp.float32)
        return ()

    lax.fori_loop(0, C, zb, ())
    for kblk in range(R // C):
        pltpu.sync_copy(rows.at[0], acc.at[pl.ds(sid * R + kblk * C, C)])

    enter_block(0)
    g_desc = {0: gather(0), 1: gather(1)}
    plsc.subcore_barrier()

    for q in range(K):
        t = q + 2
        if t < K and t % IB == 0:
            enter_block(t // IB)
        g_desc.pop(q).wait()
        b, jj = divmod(q, IB)
        pltpu.sync_copy(rows.at[q % 2], acc.at[edg.at[b % 3, jj, 1]],
                        add=True)
        if t < K:
            g_desc[t] = gather(t)

    plsc.subcore_barrier()
    flush = [
        pltpu.async_copy(acc.at[pl.ds(sid * R + kblk * C, C)],
                         parts.at[cid, pl.ds(sid * R + kblk * C, C)], semf)
        for kblk in range(R // C)
    ]
    for d2 in flush:
        d2.wait()


_gs_kernel = functools.partial(
    pl.kernel,
    out_type=jax.ShapeDtypeStruct((NC, N_PAD, D), jnp.float32),
    mesh=_MESH,
    scratch_types=[
        pltpu.VMEM((3, IB, 2, C), jnp.int32),
        pltpu.VMEM((2, C, D), jnp.float32),
        pltpu.SemaphoreType.DMA,
        pltpu.SemaphoreType.DMA,
        pltpu.SemaphoreType.DMA,
        pltpu.SemaphoreType.DMA,
        pltpu.VMEM_SHARED((N_PAD, D), jnp.float32),
    ],
)(_gs_body)


def _dinv_col(d_ref):
    t = lax.transpose(d_ref[...], (1, 0))
    deg = t[:, 0:1] + t[:, 1:2] + 1.0
    return lax.rsqrt(deg)


def _mm_body(d_ref, x_ref, w_ref, g_ref):
    h = jnp.dot(x_ref[...], w_ref[...], preferred_element_type=jnp.float32)
    g_ref[...] = h * _dinv_col(d_ref)


def _matmul_scaled(deg_parts, x, W):
    return pl.pallas_call(
        _mm_body,
        grid=(N_PAD // BLK,),
        in_specs=[
            pl.BlockSpec((NC, BLK), lambda i: (0, i)),
            pl.BlockSpec((BLK, D), lambda i: (i, 0)),
            pl.BlockSpec((D, D), lambda i: (0, 0)),
        ],
        out_specs=pl.BlockSpec((BLK, D), lambda i: (i, 0)),
        out_shape=jax.ShapeDtypeStruct((N, D), jnp.float32),
    )(deg_parts, x, W)


def _fin_body(d_ref, p_ref, g_ref, b_ref, o_ref):
    s = p_ref[0] + p_ref[1] + g_ref[...]
    o_ref[...] = s * _dinv_col(d_ref) + b_ref[...]


def _finalize(deg_parts, parts, g, b2):
    return pl.pallas_call(
        _fin_body,
        grid=(N_PAD // BLK,),
        in_specs=[
            pl.BlockSpec((NC, BLK), lambda i: (0, i)),
            pl.BlockSpec((NC, BLK, D), lambda i: (0, i, 0)),
            pl.BlockSpec((BLK, D), lambda i: (i, 0)),
            pl.BlockSpec((1, D), lambda i: (0, 0)),
        ],
        out_specs=pl.BlockSpec((BLK, D), lambda i: (i, 0)),
        out_shape=jax.ShapeDtypeStruct((N, D), jnp.float32),
    )(deg_parts, parts, g, b2)


def kernel(x, edge_index, W, b):
    deg_parts = _deg_kernel(edge_index)
    g = _matmul_scaled(deg_parts, x, W)
    parts = _gs_kernel(g, edge_index)
    return _finalize(deg_parts, parts, g, b.reshape(1, D))


# speedup vs baseline: 50.0056x; 1.0088x over previous
import functools

import jax
import jax.numpy as jnp
from jax import lax
from jax.experimental import pallas as pl
from jax.experimental.pallas import tpu as pltpu
from jax.experimental.pallas import tpu_sc as plsc

N = 10000
D = 128
E = 320000

NC = 2
NS = 16
NW = NC * NS
C = 128
NCH = E // C
K = 79
N_PAD = 10240
R = N_PAD // NS
BLK = 1024

IB = 16
NB = (K + IB - 1) // IB

_MESH = plsc.VectorSubcoreMesh(core_axis_name="c", subcore_axis_name="s")


def _chunk_off(wid, i):
    cstart = 78 * wid + jnp.minimum(wid, 4)
    if i == K - 1:
        return jnp.where(wid < 4, (cstart + (K - 1)) * C, 0)
    return (cstart + i) * C


def _dummy_fill(ref, row, wid):
    iota = lax.iota(jnp.int32, 16)
    for kk in range(C // 16):
        src_vals = (wid * 997 + kk * 1009 + iota * 131) % N
        dst_vals = N + (wid * 16 + kk * 16 + iota) % (N_PAD - N)
        ref[row, 0, pl.ds(kk * 16, 16)] = src_vals
        ref[row, 1, pl.ds(kk * 16, 16)] = dst_vals


def _deg_body(edge_hbm, deg_out, edg, ones_v, zbuf, semi, sems, acc):
    cid = lax.axis_index("c")
    sid = lax.axis_index("s")
    wid = sid * NC + cid

    descs = [
        pltpu.async_copy(edge_hbm.at[:, pl.ds(_chunk_off(wid, q), C)],
                         edg.at[q], semi)
        for q in range(K)
    ]

    for i in range(R // 16):
        zbuf[pl.ds(i * 16, 16)] = jnp.zeros((16,), jnp.float32)
    for i in range(C // 16):
        ones_v[pl.ds(i * 16, 16)] = jnp.ones((16,), jnp.float32)

    pltpu.sync_copy(zbuf, acc.at[pl.ds(sid * R, R)])
    plsc.subcore_barrier()

    for d2 in descs:
        d2.wait()

    @pl.when(wid >= 4)
    def _():
        _dummy_fill(edg, K - 1, wid)

    sdescs = [
        pltpu.async_copy(ones_v, acc.at[edg.at[q, 1]], sems, add=True)
        for q in range(K)
    ]
    for d2 in sdescs:
        d2.wait()

    plsc.subcore_barrier()
    pltpu.sync_copy(acc.at[pl.ds(sid * R, R)],
                    deg_out.at[cid, pl.ds(sid * R, R)])


_deg_kernel = functools.partial(
    pl.kernel,
    out_type=jax.ShapeDtypeStruct((NC, N_PAD), jnp.float32),
    mesh=_MESH,
    scratch_types=[
        pltpu.VMEM((K, 2, C), jnp.int32),
        pltpu.VMEM((C,), jnp.float32),
        pltpu.VMEM((R,), jnp.float32),
        pltpu.SemaphoreType.DMA,
        pltpu.SemaphoreType.DMA,
        pltpu.VMEM_SHARED((N_PAD,), jnp.float32),
    ],
)(_deg_body)


def _gs_body(g_hbm, edge_hbm, parts, edg, rows,
             semi, semg0, semg1, semf, acc):
    cid = lax.axis_index("c")
    sid = lax.axis_index("s")
    wid = sid * NC + cid
    semg = (semg0, semg1)

    idx_pend = {}

    def issue_idx(b):
        slot = b % 3
        nrows = IB if b < NB - 1 else (K - b * IB)
        idx_pend[b] = [
            pltpu.async_copy(
                edge_hbm.at[:, pl.ds(_chunk_off(wid, b * IB + i), C)],
                edg.at[slot, i], semi)
            for i in range(nrows)
        ]

    def enter_block(b):
        for d2 in idx_pend.pop(b):
            d2.wait()
        if b == NB - 1:
            @pl.when(wid >= 4)
            def _():
                _dummy_fill(edg.at[b % 3], (K - 1) % IB, wid)
        if b + 1 < NB:
            issue_idx(b + 1)

    def gather(q):
        b, jj = divmod(q, IB)
        return pltpu.async_copy(g_hbm.at[edg.at[b % 3, jj, 0]],
                                rows.at[q % 2], semg[q % 2])

    issue_idx(0)

    def zb(i, _):
        for l in range(D // 16):
            rows[0, i, pl.ds(l * 16, 16)] = jnp.zeros((16,), jnp.float32)
        return ()

    lax.fori_loop(0, C, zb, ())
    zdesc = [
        pltpu.async_copy(rows.at[0], acc.at[pl.ds(sid * R + kblk * C, C)],
                         semf)
        for kblk in range(R // C)
    ]
    enter_block(0)
    g_desc = {1: gather(1)}
    for d2 in zdesc:
        d2.wait()
    g_desc[0] = gather(0)
    plsc.subcore_barrier()

    for q in range(K):
        t = q + 2
        if t < K and t % IB == 0:
            enter_block(t // IB)
        g_desc.pop(q).wait()
        b, jj = divmod(q, IB)
        pltpu.sync_copy(rows.at[q % 2], acc.at[edg.at[b % 3, jj, 1]],
                        add=True)
        if t < K:
            g_desc[t] = gather(t)

    plsc.subcore_barrier()
    flush = [
        pltpu.async_copy(acc.at[pl.ds(sid * R + kblk * C, C)],
                         parts.at[cid, pl.ds(sid * R + kblk * C, C)], semf)
        for kblk in range(R // C)
    ]
    for d2 in flush:
        d2.wait()


_gs_kernel = functools.partial(
    pl.kernel,
    out_type=jax.ShapeDtypeStruct((NC, N_PAD, D), jnp.float32),
    mesh=_MESH,
    scratch_types=[
        pltpu.VMEM((3, IB, 2, C), jnp.int32),
        pltpu.VMEM((2, C, D), jnp.float32),
        pltpu.SemaphoreType.DMA,
        pltpu.SemaphoreType.DMA,
        pltpu.SemaphoreType.DMA,
        pltpu.SemaphoreType.DMA,
        pltpu.VMEM_SHARED((N_PAD, D), jnp.float32),
    ],
)(_gs_body)


def _dinv_col(d_ref):
    t = lax.transpose(d_ref[...], (1, 0))
    deg = t[:, 0:1] + t[:, 1:2] + 1.0
    return lax.rsqrt(deg)


def _mm_body(d_ref, x_ref, w_ref, g_ref):
    h = jnp.dot(x_ref[...], w_ref[...], preferred_element_type=jnp.float32)
    g_ref[...] = h * _dinv_col(d_ref)


def _matmul_scaled(deg_parts, x, W):
    return pl.pallas_call(
        _mm_body,
        grid=(N_PAD // BLK,),
        in_specs=[
            pl.BlockSpec((NC, BLK), lambda i: (0, i)),
            pl.BlockSpec((BLK, D), lambda i: (i, 0)),
            pl.BlockSpec((D, D), lambda i: (0, 0)),
        ],
        out_specs=pl.BlockSpec((BLK, D), lambda i: (i, 0)),
        out_shape=jax.ShapeDtypeStruct((N, D), jnp.float32),
    )(deg_parts, x, W)


def _fin_body(d_ref, p_ref, g_ref, b_ref, o_ref):
    s = p_ref[0] + p_ref[1] + g_ref[...]
    o_ref[...] = s * _dinv_col(d_ref) + b_ref[...]


def _finalize(deg_parts, parts, g, b2):
    return pl.pallas_call(
        _fin_body,
        grid=(N_PAD // BLK,),
        in_specs=[
            pl.BlockSpec((NC, BLK), lambda i: (0, i)),
            pl.BlockSpec((NC, BLK, D), lambda i: (0, i, 0)),
            pl.BlockSpec((BLK, D), lambda i: (i, 0)),
            pl.BlockSpec((1, D), lambda i: (0, 0)),
        ],
        out_specs=pl.BlockSpec((BLK, D), lambda i: (i, 0)),
        out_shape=jax.ShapeDtypeStruct((N, D), jnp.float32),
    )(deg_parts, parts, g, b2)


def kernel(x, edge_index, W, b):
    deg_parts = _deg_kernel(edge_index)
    g = _matmul_scaled(deg_parts, x, W)
    parts = _gs_kernel(g, edge_index)
    return _finalize(deg_parts, parts, g, b.reshape(1, D))
